# Initial kernel scaffold; baseline (speedup 1.0000x reference)
import jax
import jax.numpy as jnp
from jax.experimental import pallas as pl


def kernel(x, edge_index, edge_weight, W1, W2):
    raise NotImplementedError("write your pallas kernel here")


# speedup vs baseline: 2.3941x; 2.3941x over previous
import functools

import jax
import jax.numpy as jnp
from jax import lax
from jax.experimental import pallas as pl
from jax.experimental.pallas import tpu as pltpu
from jax.experimental.pallas import tpu_sc as plsc

N_NODES = 10000
D_FEAT = 256
HIDDEN = 256
OUT_DIM = 64
N_EDGES = 160000
EPSILON = 0.1

NC = 2
NS = 16
L = 16
D = 128

_K = 80

_EPT2 = 5120
_E_PAD = NC * NS * _EPT2


_ROWS_BLK = 1000


def _mm1_body(x_ref, w_ref, o_ref):
    o_ref[0] = jnp.dot(x_ref[...], w_ref[...],
                       preferred_element_type=jnp.float32)


def _matmul_split(x, W1):
    n = x.shape[0]
    grid = (NC, n // _ROWS_BLK)
    return pl.pallas_call(
        _mm1_body,
        grid=grid,
        in_specs=[
            pl.BlockSpec((_ROWS_BLK, D_FEAT), lambda c, i: (i, 0)),
            pl.BlockSpec((D_FEAT, HIDDEN // NC), lambda c, i: (0, c)),
        ],
        out_specs=pl.BlockSpec((1, _ROWS_BLK, HIDDEN // NC),
                               lambda c, i: (c, i, 0)),
        out_shape=jax.ShapeDtypeStruct((NC, n, HIDDEN // NC), jnp.float32),
    )(x, W1)


def _mm2_body(p_ref, x_ref, w_ref, o_ref):
    h = HIDDEN // NC
    ha = (1.0 - EPSILON) * jnp.maximum(p_ref[0], 0.0) + EPSILON * x_ref[:, :h]
    hb = (1.0 - EPSILON) * jnp.maximum(p_ref[1], 0.0) + EPSILON * x_ref[:, h:]
    o = (jnp.dot(ha, w_ref[:h], preferred_element_type=jnp.float32)
         + jnp.dot(hb, w_ref[h:], preferred_element_type=jnp.float32))
    o_ref[:, :OUT_DIM] = o
    o_ref[:, OUT_DIM:] = jnp.zeros((o.shape[0], D - OUT_DIM), jnp.float32)


def _blend_matmul(p1, x, W2):
    n = x.shape[0]
    grid = (n // _ROWS_BLK,)
    return pl.pallas_call(
        _mm2_body,
        grid=grid,
        in_specs=[
            pl.BlockSpec((NC, _ROWS_BLK, HIDDEN // NC), lambda i: (0, i, 0)),
            pl.BlockSpec((_ROWS_BLK, D_FEAT), lambda i: (i, 0)),
            pl.BlockSpec((HIDDEN, OUT_DIM), lambda i: (0, 0)),
        ],
        out_specs=pl.BlockSpec((_ROWS_BLK, D), lambda i: (i, 0)),
        out_shape=jax.ShapeDtypeStruct((n, D), jnp.float32),
    )(p1, x, W2)


def _sum_body(a_ref, o_ref):
    o_ref[...] = a_ref[0, :, :OUT_DIM] + a_ref[1, :, :OUT_DIM]


def _sum_partials(p2):
    n = p2.shape[1]
    return pl.pallas_call(
        _sum_body,
        grid=(n // _ROWS_BLK,),
        in_specs=[pl.BlockSpec((NC, _ROWS_BLK, D), lambda i: (0, i, 0))],
        out_specs=pl.BlockSpec((_ROWS_BLK, OUT_DIM), lambda i: (i, 0)),
        out_shape=jax.ShapeDtypeStruct((n, OUT_DIM), jnp.float32),
    )(p2)


def _make_propagate(ept, feature_split, scale_regs):
    nb = ept // _K
    assert nb * _K == ept
    rpt = 624
    tail = N_NODES - NS * rpt
    zrows = 104

    mesh = plsc.VectorSubcoreMesh(core_axis_name="c", subcore_axis_name="s",
                                  num_cores=NC, num_subcores=NS)

    @functools.partial(
        pl.kernel,
        out_type=jax.ShapeDtypeStruct((NC, N_NODES, D), jnp.float32),
        mesh=mesh,
        scratch_types=[
            pltpu.VMEM((_K,), jnp.int32),
            pltpu.VMEM((_K,), jnp.int32),
            pltpu.VMEM((_K,), jnp.float32),
            pltpu.VMEM((_K, D), jnp.float32),
            pltpu.VMEM((zrows, D), jnp.float32),
            pltpu.VMEM_SHARED((N_NODES, D), jnp.float32),
            pltpu.SemaphoreType.DMA,
        ],
    )
    def prop(h_hbm, src_hbm, dst_hbm, w_hbm, out_hbm,
             srcv, dstv, wv, rows, zbuf, acc, sem):
        c = lax.axis_index("c")
        s = lax.axis_index("s")

        zero = jnp.zeros((L,), jnp.float32)

        def zero_body(r, _):
            for j in range(D // L):
                zbuf[r, pl.ds(j * L, L)] = zero
            return 0

        lax.fori_loop(0, zrows, zero_body, 0)
        for z in range(rpt // zrows):
            pltpu.sync_copy(zbuf, acc.at[pl.ds(s * rpt + z * zrows, zrows)])

        @pl.when(s == 0)
        def _zero_tail():
            pltpu.sync_copy(zbuf.at[pl.ds(0, tail)],
                            acc.at[pl.ds(NS * rpt, tail)])

        plsc.subcore_barrier()

        if feature_split:
            tile_base = s * ept
            row_off = c * N_NODES
        else:
            tile_base = (c * NS + s) * ept
            row_off = None

        def batch_body(b, _):
            off = tile_base + b * _K
            pltpu.sync_copy(src_hbm.at[pl.ds(off, _K)], srcv)
            pltpu.sync_copy(dst_hbm.at[pl.ds(off, _K)], dstv)
            pltpu.sync_copy(w_hbm.at[pl.ds(off, _K)], wv)
            if row_off is not None:
                roff = jnp.full((L,), row_off, jnp.int32)
                for j in range(_K // L):
                    srcv[pl.ds(j * L, L)] = srcv[pl.ds(j * L, L)] + roff
            pltpu.async_copy(h_hbm.at[srcv], rows, sem).wait()

            def scale_body(kg, _):
                wvec = wv[pl.ds(kg * L, L)]
                for t in range(L):
                    wb = jnp.full((L,), wvec[t])
                    k = kg * L + t
                    for j in range(scale_regs):
                        rows[k, pl.ds(j * L, L)] = (
                            rows[k, pl.ds(j * L, L)] * wb)
                return 0

            lax.fori_loop(0, _K // L, scale_body, 0)
            pltpu.sync_copy(rows, acc.at[dstv], add=True)
            return 0

        lax.fori_loop(0, nb, batch_body, 0)
        plsc.subcore_barrier()

        pltpu.sync_copy(acc.at[pl.ds(s * rpt, rpt)],
                        out_hbm.at[c, pl.ds(s * rpt, rpt)])

        @pl.when(s == 0)
        def _write_tail():
            pltpu.sync_copy(acc.at[pl.ds(NS * rpt, tail)],
                            out_hbm.at[c, pl.ds(NS * rpt, tail)])

    return prop


_prop_hidden = _make_propagate(N_EDGES // NS, True, D // L)
_prop_out = _make_propagate(_EPT2, False, OUT_DIM // L)


def kernel(x, edge_index, edge_weight, W1, W2):
    src = edge_index[0].astype(jnp.int32)
    dst = edge_index[1].astype(jnp.int32)
    w = edge_weight.astype(jnp.float32)

    pad = _E_PAD - src.shape[0]
    zpad_i = jnp.zeros((pad,), jnp.int32)
    src_p = jnp.concatenate([src, zpad_i])
    dst_p = jnp.concatenate([dst, zpad_i])
    w_p = jnp.concatenate([w, jnp.zeros((pad,), jnp.float32)])

    h0 = _matmul_split(x, W1)
    h0_flat = h0.reshape(NC * N_NODES, HIDDEN // NC)
    p1 = _prop_hidden(h0_flat, src, dst, w)
    h2 = _blend_matmul(p1, x, W2)
    p2 = _prop_out(h2, src_p, dst_p, w_p)
    return _sum_partials(p2)


# speedup vs baseline: 3.5436x; 1.4801x over previous
import functools

import jax
import jax.numpy as jnp
from jax import lax
from jax.experimental import pallas as pl
from jax.experimental.pallas import tpu as pltpu
from jax.experimental.pallas import tpu_sc as plsc

N_NODES = 10000
D_FEAT = 256
HIDDEN = 256
OUT_DIM = 64
N_EDGES = 160000
EPSILON = 0.1

NC = 2
NS = 16
L = 16
D = 128

_K = 128
_EROWS = 1280
_E_PAD = _EROWS * _K


_ROWS_BLK = 1000


def _mm1_body(x_ref, w_ref, o_ref):
    o_ref[0] = jnp.dot(x_ref[...], w_ref[...],
                       preferred_element_type=jnp.float32)


def _matmul_split(x, W1):
    n = x.shape[0]
    grid = (NC, n // _ROWS_BLK)
    return pl.pallas_call(
        _mm1_body,
        grid=grid,
        in_specs=[
            pl.BlockSpec((_ROWS_BLK, D_FEAT), lambda c, i: (i, 0)),
            pl.BlockSpec((D_FEAT, HIDDEN // NC), lambda c, i: (0, c)),
        ],
        out_specs=pl.BlockSpec((1, _ROWS_BLK, HIDDEN // NC),
                               lambda c, i: (c, i, 0)),
        out_shape=jax.ShapeDtypeStruct((NC, n, HIDDEN // NC), jnp.float32),
    )(x, W1)


def _mm2_body(p_ref, x_ref, w_ref, o_ref):
    h = HIDDEN // NC
    ha = (1.0 - EPSILON) * jnp.maximum(p_ref[0], 0.0) + EPSILON * x_ref[:, :h]
    hb = (1.0 - EPSILON) * jnp.maximum(p_ref[1], 0.0) + EPSILON * x_ref[:, h:]
    o = (jnp.dot(ha, w_ref[:h], preferred_element_type=jnp.float32)
         + jnp.dot(hb, w_ref[h:], preferred_element_type=jnp.float32))
    o_ref[:, :OUT_DIM] = o
    o_ref[:, OUT_DIM:] = jnp.zeros((o.shape[0], D - OUT_DIM), jnp.float32)


def _blend_matmul(p1, x, W2):
    n = x.shape[0]
    grid = (n // _ROWS_BLK,)
    return pl.pallas_call(
        _mm2_body,
        grid=grid,
        in_specs=[
            pl.BlockSpec((NC, _ROWS_BLK, HIDDEN // NC), lambda i: (0, i, 0)),
            pl.BlockSpec((_ROWS_BLK, D_FEAT), lambda i: (i, 0)),
            pl.BlockSpec((HIDDEN, OUT_DIM), lambda i: (0, 0)),
        ],
        out_specs=pl.BlockSpec((_ROWS_BLK, D), lambda i: (i, 0)),
        out_shape=jax.ShapeDtypeStruct((n, D), jnp.float32),
    )(p1, x, W2)


def _sum_body(a_ref, o_ref):
    o_ref[...] = a_ref[0, :, :OUT_DIM] + a_ref[1, :, :OUT_DIM]


def _sum_partials(p2):
    n = p2.shape[1]
    return pl.pallas_call(
        _sum_body,
        grid=(n // _ROWS_BLK,),
        in_specs=[pl.BlockSpec((NC, _ROWS_BLK, D), lambda i: (0, i, 0))],
        out_specs=pl.BlockSpec((_ROWS_BLK, OUT_DIM), lambda i: (i, 0)),
        out_shape=jax.ShapeDtypeStruct((n, OUT_DIM), jnp.float32),
    )(p2)


def _make_propagate(feature_split, scale_regs):
    nb = _EROWS // NS if feature_split else _EROWS // (NC * NS)
    nbc = 40 if feature_split else 8
    nch = nb // nbc
    assert nbc % 8 == 0 and nbc % 2 == 0 and nbc * nch == nb
    rpt = 624
    tail = N_NODES - NS * rpt
    zrows = 104

    mesh = plsc.VectorSubcoreMesh(core_axis_name="c", subcore_axis_name="s",
                                  num_cores=NC, num_subcores=NS)

    @functools.partial(
        pl.kernel,
        out_type=jax.ShapeDtypeStruct((NC, N_NODES, D), jnp.float32),
        mesh=mesh,
        scratch_types=[
            pltpu.VMEM((nbc, _K), jnp.int32),
            pltpu.VMEM((nbc, _K), jnp.int32),
            pltpu.VMEM((nbc, _K), jnp.float32),
            pltpu.VMEM((_K, D), jnp.float32),
            pltpu.VMEM((_K, D), jnp.float32),
            pltpu.VMEM_SHARED((N_NODES, D), jnp.float32),
            pltpu.SemaphoreType.DMA,
            pltpu.SemaphoreType.DMA,
        ],
    )
    def prop(h_hbm, src_hbm, dst_hbm, w_hbm, out_hbm,
             srcv, dstv, wv, rows0, rows1, acc, sem0, sem1):
        c = lax.axis_index("c")
        s = lax.axis_index("s")
        tile_idx = s if feature_split else c * NS + s
        row_base = tile_idx * nb

        zero = jnp.zeros((L,), jnp.float32)

        def zero_body(r, _):
            for j in range(D // L):
                rows0[r, pl.ds(j * L, L)] = zero
            return 0

        lax.fori_loop(0, zrows, zero_body, 0)
        for z in range(rpt // zrows):
            pltpu.sync_copy(rows0.at[pl.ds(0, zrows)],
                            acc.at[pl.ds(s * rpt + z * zrows, zrows)])

        @pl.when(s == 0)
        def _zero_tail():
            pltpu.sync_copy(rows0.at[pl.ds(0, tail)],
                            acc.at[pl.ds(NS * rpt, tail)])

        plsc.subcore_barrier()

        def scale(rows, b):
            def scale_body(kg, _):
                wvec = wv[b, pl.ds(kg * L, L)]
                for t in range(L):
                    wb = jnp.full((L,), wvec[t])
                    k = kg * L + t
                    for j in range(scale_regs):
                        rows[k, pl.ds(j * L, L)] = (
                            rows[k, pl.ds(j * L, L)] * wb)
                return 0

            lax.fori_loop(0, _K // L, scale_body, 0)

        for ch in range(nch):
            base = row_base + ch * nbc
            pltpu.sync_copy(src_hbm.at[pl.ds(base, nbc)], srcv)
            pltpu.sync_copy(dst_hbm.at[pl.ds(base, nbc)], dstv)
            pltpu.sync_copy(w_hbm.at[pl.ds(base, nbc)], wv)

            if feature_split:
                roff = jnp.full((L,), c * N_NODES, jnp.int32)

                def off_body(r, _):
                    for j in range(_K // L):
                        srcv[r, pl.ds(j * L, L)] = (
                            srcv[r, pl.ds(j * L, L)] + roff)
                    return 0

                lax.fori_loop(0, nbc, off_body, 0)

            pltpu.async_copy(h_hbm.at[srcv.at[0]], rows0, sem0)

            def batch_body(b2, _):
                b0 = 2 * b2
                b1 = b0 + 1
                pltpu.async_copy(h_hbm.at[srcv.at[b1]], rows1, sem1)
                pltpu.make_async_copy(h_hbm.at[srcv.at[b0]], rows0,
                                      sem0).wait()
                scale(rows0, b0)
                pltpu.sync_copy(rows0, acc.at[dstv.at[b0]], add=True)
                bn = jnp.minimum(b0 + 2, nbc - 2)
                pltpu.async_copy(h_hbm.at[srcv.at[bn]], rows0, sem0)
                pltpu.make_async_copy(h_hbm.at[srcv.at[b1]], rows1,
                                      sem1).wait()
                scale(rows1, b1)
                pltpu.sync_copy(rows1, acc.at[dstv.at[b1]], add=True)
                return 0

            lax.fori_loop(0, nbc // 2, batch_body, 0)
            pltpu.make_async_copy(h_hbm.at[srcv.at[0]], rows0, sem0).wait()

        plsc.subcore_barrier()

        pltpu.sync_copy(acc.at[pl.ds(s * rpt, rpt)],
                        out_hbm.at[c, pl.ds(s * rpt, rpt)])

        @pl.when(s == 0)
        def _write_tail():
            pltpu.sync_copy(acc.at[pl.ds(NS * rpt, tail)],
                            out_hbm.at[c, pl.ds(NS * rpt, tail)])

    return prop


_prop_hidden = _make_propagate(True, D // L)
_prop_out = _make_propagate(False, OUT_DIM // L)


def kernel(x, edge_index, edge_weight, W1, W2):
    src = edge_index[0].astype(jnp.int32)
    dst = edge_index[1].astype(jnp.int32)
    w = edge_weight.astype(jnp.float32)

    pad = _E_PAD - src.shape[0]
    zpad_i = jnp.zeros((pad,), jnp.int32)
    src_p = jnp.concatenate([src, zpad_i]).reshape(_EROWS, _K)
    dst_p = jnp.concatenate([dst, zpad_i]).reshape(_EROWS, _K)
    w_p = jnp.concatenate(
        [w, jnp.zeros((pad,), jnp.float32)]).reshape(_EROWS, _K)

    h0 = _matmul_split(x, W1)
    h0_flat = h0.reshape(NC * N_NODES, HIDDEN // NC)
    p1 = _prop_hidden(h0_flat, src_p, dst_p, w_p)
    h2 = _blend_matmul(p1, x, W2)
    p2 = _prop_out(h2, src_p, dst_p, w_p)
    return _sum_partials(p2)


# speedup vs baseline: 7.2117x; 2.0351x over previous
import functools

import jax
import jax.numpy as jnp
from jax import lax
from jax.experimental import pallas as pl
from jax.experimental.pallas import tpu as pltpu
from jax.experimental.pallas import tpu_sc as plsc

N_NODES = 10000
D_FEAT = 256
HIDDEN = 256
OUT_DIM = 64
N_EDGES = 160000
EPSILON = 0.1

NC = 2
NS = 16
L = 16
D = 128

_K = 128
_EROWS = 1280
_E_PAD = _EROWS * _K


_ROWS_BLK = 1000


def _mm1_body(x_ref, w_ref, o_ref):
    o_ref[0] = jnp.dot(x_ref[...], w_ref[...],
                       preferred_element_type=jnp.float32)


def _matmul_split(x, W1):
    n = x.shape[0]
    grid = (NC, n // _ROWS_BLK)
    return pl.pallas_call(
        _mm1_body,
        grid=grid,
        in_specs=[
            pl.BlockSpec((_ROWS_BLK, D_FEAT), lambda c, i: (i, 0)),
            pl.BlockSpec((D_FEAT, HIDDEN // NC), lambda c, i: (0, c)),
        ],
        out_specs=pl.BlockSpec((1, _ROWS_BLK, HIDDEN // NC),
                               lambda c, i: (c, i, 0)),
        out_shape=jax.ShapeDtypeStruct((NC, n, HIDDEN // NC), jnp.float32),
    )(x, W1)


def _mm2_body(p_ref, x_ref, w_ref, o_ref):
    h = HIDDEN // NC
    ha = (1.0 - EPSILON) * jnp.maximum(p_ref[0], 0.0) + EPSILON * x_ref[:, :h]
    hb = (1.0 - EPSILON) * jnp.maximum(p_ref[1], 0.0) + EPSILON * x_ref[:, h:]
    o = (jnp.dot(ha, w_ref[:h], preferred_element_type=jnp.float32)
         + jnp.dot(hb, w_ref[h:], preferred_element_type=jnp.float32))
    o_ref[:, :OUT_DIM] = o
    o_ref[:, OUT_DIM:] = jnp.zeros((o.shape[0], D - OUT_DIM), jnp.float32)


def _blend_matmul(p1, x, W2):
    n = x.shape[0]
    grid = (n // _ROWS_BLK,)
    return pl.pallas_call(
        _mm2_body,
        grid=grid,
        in_specs=[
            pl.BlockSpec((NC, _ROWS_BLK, HIDDEN // NC), lambda i: (0, i, 0)),
            pl.BlockSpec((_ROWS_BLK, D_FEAT), lambda i: (i, 0)),
            pl.BlockSpec((HIDDEN, OUT_DIM), lambda i: (0, 0)),
        ],
        out_specs=pl.BlockSpec((_ROWS_BLK, D), lambda i: (i, 0)),
        out_shape=jax.ShapeDtypeStruct((n, D), jnp.float32),
    )(p1, x, W2)


def _sum_body(a_ref, o_ref):
    o_ref[...] = a_ref[0, :, :OUT_DIM] + a_ref[1, :, :OUT_DIM]


def _sum_partials(p2):
    n = p2.shape[1]
    return pl.pallas_call(
        _sum_body,
        grid=(n // _ROWS_BLK,),
        in_specs=[pl.BlockSpec((NC, _ROWS_BLK, D), lambda i: (0, i, 0))],
        out_specs=pl.BlockSpec((_ROWS_BLK, OUT_DIM), lambda i: (i, 0)),
        out_shape=jax.ShapeDtypeStruct((n, OUT_DIM), jnp.float32),
    )(p2)


def _make_propagate(feature_split, scale_regs):
    nb = _EROWS // NS if feature_split else _EROWS // (NC * NS)
    nbc = 40 if feature_split else 8
    nch = nb // nbc
    assert nbc % 8 == 0 and nbc % 2 == 0 and nbc * nch == nb
    rpt = 624
    tail = N_NODES - NS * rpt
    zrows = 104

    mesh = plsc.VectorSubcoreMesh(core_axis_name="c", subcore_axis_name="s",
                                  num_cores=NC, num_subcores=NS)

    @functools.partial(
        pl.kernel,
        out_type=jax.ShapeDtypeStruct((NC, N_NODES, D), jnp.float32),
        mesh=mesh,
        scratch_types=[
            pltpu.VMEM((nbc, _K), jnp.int32),
            pltpu.VMEM((nbc, _K), jnp.int32),
            pltpu.VMEM((nbc, _K), jnp.float32),
            pltpu.VMEM((_K, D), jnp.float32),
            pltpu.VMEM((_K, D), jnp.float32),
            pltpu.VMEM_SHARED((N_NODES, D), jnp.float32),
            pltpu.SemaphoreType.DMA,
            pltpu.SemaphoreType.DMA,
        ],
    )
    def prop(h_hbm, src_hbm, dst_hbm, w_hbm, out_hbm,
             srcv, dstv, wv, rows0, rows1, acc, sem0, sem1):
        c = lax.axis_index("c")
        s = lax.axis_index("s")
        tile_idx = s if feature_split else c * NS + s
        row_base = tile_idx * nb

        zero = jnp.zeros((L,), jnp.float32)

        def zero_body(r, _):
            for j in range(D // L):
                rows0[r, pl.ds(j * L, L)] = zero
            return 0

        lax.fori_loop(0, zrows, zero_body, 0)
        for z in range(rpt // zrows):
            pltpu.sync_copy(rows0.at[pl.ds(0, zrows)],
                            acc.at[pl.ds(s * rpt + z * zrows, zrows)])

        @pl.when(s == 0)
        def _zero_tail():
            pltpu.sync_copy(rows0.at[pl.ds(0, tail)],
                            acc.at[pl.ds(NS * rpt, tail)])

        plsc.subcore_barrier()

        def scale(rows, b):
            def scale_body(kg, _):
                wvec = wv[b, pl.ds(kg * L, L)]
                for t in range(L):
                    wb = jnp.full((L,), wvec[t])
                    k = kg * L + t
                    for j in range(scale_regs):
                        rows[k, pl.ds(j * L, L)] = (
                            rows[k, pl.ds(j * L, L)] * wb)
                return 0

            lax.fori_loop(0, _K // L, scale_body, 0)

        for ch in range(nch):
            base = row_base + ch * nbc
            pltpu.sync_copy(src_hbm.at[pl.ds(base, nbc)], srcv)
            pltpu.sync_copy(dst_hbm.at[pl.ds(base, nbc)], dstv)
            pltpu.sync_copy(w_hbm.at[pl.ds(base, nbc)], wv)

            if feature_split:
                roff = jnp.full((L,), c * N_NODES, jnp.int32)

                def off_body(r, _):
                    for j in range(_K // L):
                        srcv[r, pl.ds(j * L, L)] = (
                            srcv[r, pl.ds(j * L, L)] + roff)
                    return 0

                lax.fori_loop(0, nbc, off_body, 0)

            pltpu.async_copy(h_hbm.at[srcv.at[0]], rows0, sem0)

            def batch_body(b2, _):
                b0 = 2 * b2
                b1 = b0 + 1
                pltpu.async_copy(h_hbm.at[srcv.at[b1]], rows1, sem1)
                pltpu.make_async_copy(h_hbm.at[srcv.at[b0]], rows0,
                                      sem0).wait()
                scale(rows0, b0)
                pltpu.sync_copy(rows0, acc.at[dstv.at[b0]], add=True)
                bn = jnp.minimum(b0 + 2, nbc - 2)
                pltpu.async_copy(h_hbm.at[srcv.at[bn]], rows0, sem0)
                pltpu.make_async_copy(h_hbm.at[srcv.at[b1]], rows1,
                                      sem1).wait()
                scale(rows1, b1)
                pltpu.sync_copy(rows1, acc.at[dstv.at[b1]], add=True)
                return 0

            lax.fori_loop(0, nbc // 2, batch_body, 0)
            pltpu.make_async_copy(h_hbm.at[srcv.at[0]], rows0, sem0).wait()

        plsc.subcore_barrier()

        pltpu.sync_copy(acc.at[pl.ds(s * rpt, rpt)],
                        out_hbm.at[c, pl.ds(s * rpt, rpt)])

        @pl.when(s == 0)
        def _write_tail():
            pltpu.sync_copy(acc.at[pl.ds(NS * rpt, tail)],
                            out_hbm.at[c, pl.ds(NS * rpt, tail)])

    return prop


_prop_hidden = _make_propagate(True, D // L)
_prop_out = _make_propagate(False, OUT_DIM // L)


def kernel(x, edge_index, edge_weight, W1, W2):
    src = edge_index[0].astype(jnp.int32)
    dst = edge_index[1].astype(jnp.int32)
    w = edge_weight.astype(jnp.float32)

    pad = _E_PAD - src.shape[0]
    spread = jnp.arange(pad, dtype=jnp.int32) % N_NODES
    src_p = jnp.concatenate([src, spread]).reshape(_EROWS, _K)
    dst_p = jnp.concatenate([dst, spread]).reshape(_EROWS, _K)
    w_p = jnp.concatenate(
        [w, jnp.zeros((pad,), jnp.float32)]).reshape(_EROWS, _K)

    h0 = _matmul_split(x, W1)
    h0_flat = h0.reshape(NC * N_NODES, HIDDEN // NC)
    p1 = _prop_hidden(h0_flat, src_p, dst_p, w_p)
    h2 = _blend_matmul(p1, x, W2)
    p2 = _prop_out(h2, src_p, dst_p, w_p)
    return _sum_partials(p2)
